# baseline (speedup 1.0000x reference)
import functools

import numpy as np

import jax
import jax.numpy as jnp
from jax import lax
from jax.experimental import pallas as pl
from jax.experimental.pallas import tpu as pltpu
from jax.experimental.pallas import tpu_sc as plsc

B = 4096
D = 56
P = 18
NBINS = P + 1
HIGH_PROB = 0.3
LOW_PROB = 0.05
ACT_THR = 0.01

NC = 2
NS = 16
R1 = B // NS
R2 = B // (NC * NS)
UW = R2 * P
RANK_LO = 3072
RANK_HI = 3073


def _sc_body(x_hbm, u_hbm, out_hbm, xbuf, ubuf, mbuf, abuf, cntbuf, gathbuf,
             shared):
    c = lax.axis_index("c")
    s = lax.axis_index("s")
    row0 = s * R1
    prow = c * R2
    grow = row0 + prow

    pltpu.sync_copy(u_hbm.at[pl.ds((s * NC + c) * UW, UW)], ubuf)
    pltpu.sync_copy(x_hbm.at[pl.ds(row0, R1)], xbuf)

    iot = lax.iota(jnp.int32, 16)
    act_thr = jnp.float32(ACT_THR)
    colv = [jnp.full((16,), col, jnp.int32) for col in range(D)]

    acc = [jnp.zeros((16,), jnp.int32) for _ in range(NBINS)]
    for g in range(R1 // 16):
        rows = iot + g * 16
        m = jnp.zeros((16,), jnp.int32)
        actbits = jnp.zeros((16,), jnp.int32)
        for p in range(P):
            xv = plsc.load_gather(xbuf, [rows, colv[2 + 3 * p]])
            a = xv > act_thr
            m = m + a.astype(jnp.int32)
            actbits = actbits | jnp.where(a, jnp.int32(1 << p), 0)
        mbuf[pl.ds(g * 16, 16)] = m
        abuf[pl.ds(g * 16, 16)] = actbits
        for k in range(NBINS):
            acc[k] = acc[k] + plsc.all_reduce_population_count(m <= k)

    cnt_lo = jnp.zeros((16,), jnp.int32)
    cnt_hi = jnp.zeros((16,), jnp.int32)
    for k in range(16):
        cnt_lo = jnp.where(iot == k, acc[k], cnt_lo)
    for k in range(16, NBINS):
        cnt_hi = jnp.where(iot == (k - 16), acc[k], cnt_hi)
    cntbuf[pl.ds(0, 16)] = cnt_lo
    cntbuf[pl.ds(16, 16)] = cnt_hi

    pltpu.sync_copy(cntbuf, shared.at[pl.ds(s * 32, 32)])
    plsc.subcore_barrier()
    pltpu.sync_copy(shared, gathbuf)
    tot_lo = jnp.zeros((16,), jnp.int32)
    tot_hi = jnp.zeros((16,), jnp.int32)
    for t in range(NS):
        tot_lo = tot_lo + gathbuf[pl.ds(t * 32, 16)]
        tot_hi = tot_hi + gathbuf[pl.ds(t * 32 + 16, 16)]

    hi_valid = iot < (NBINS - 16)

    def order_stat(rank):
        lo = jnp.sum((tot_lo < rank).astype(jnp.int32))
        hi = jnp.sum(((tot_hi < rank) & hi_valid).astype(jnp.int32))
        return lo + hi

    s_lo = order_stat(RANK_LO)
    s_hi = order_stat(RANK_HI)
    thr = s_lo.astype(jnp.float32) + jnp.float32(0.25) * (
        s_hi - s_lo).astype(jnp.float32)

    high = jnp.float32(HIGH_PROB)
    low = jnp.float32(LOW_PROB)
    zero16 = jnp.zeros((16,), jnp.float32)
    for g in range(R2 // 16):
        xrows = iot + (g * 16) + prow
        m = mbuf[pl.ds(prow + g * 16, 16)]
        actbits = abuf[pl.ds(prow + g * 16, 16)]
        probv = jnp.where(m.astype(jnp.float32) > thr, high, low)
        for p in range(P):
            col = 2 + 3 * p
            uv = ubuf[pl.ds(p * R2 + g * 16, 16)]
            act = (actbits & jnp.int32(1 << p)) != 0
            mk = act & (uv < probv)
            any_masked = jnp.sum(mk.astype(jnp.int32)) > 0

            @pl.when(any_masked)
            def _(mk=mk, xrows=xrows, col=col):
                for j in (0, 1, 2):
                    plsc.store_scatter(xbuf, [xrows, colv[col + j]], zero16,
                                       mask=mk)

    pltpu.sync_copy(xbuf.at[pl.ds(prow, R2)], out_hbm.at[pl.ds(grow, R2)])


@functools.partial(
    pl.kernel,
    out_type=jax.ShapeDtypeStruct((B, D), jnp.float32),
    mesh=plsc.VectorSubcoreMesh(core_axis_name="c", subcore_axis_name="s",
                                num_cores=NC, num_subcores=NS),
    scratch_types=[
        pltpu.VMEM((R1, D), jnp.float32),
        pltpu.VMEM((UW,), jnp.float32),
        pltpu.VMEM((R1,), jnp.int32),
        pltpu.VMEM((R1,), jnp.int32),
        pltpu.VMEM((32,), jnp.int32),
        pltpu.VMEM((NS * 32,), jnp.int32),
        pltpu.VMEM_SHARED((NS * 32,), jnp.int32),
    ],
    compiler_params=pltpu.CompilerParams(needs_layout_passes=False),
)
def _masking_kernel(x_hbm, u_hbm, out_hbm, xbuf, ubuf, mbuf, abuf, cntbuf,
                    gathbuf, shared):
    _sc_body(x_hbm, u_hbm, out_hbm, xbuf, ubuf, mbuf, abuf, cntbuf, gathbuf,
             shared)


def _rotl(x, r):
    return ((x << np.uint32(r)) | (x >> np.uint32(32 - r))).astype(np.uint32)


def _threefry2x32(k0, k1, x0, x1):
    rot_a = (13, 15, 26, 6)
    rot_b = (17, 29, 16, 24)
    ks0 = np.uint32(k0)
    ks1 = np.uint32(k1)
    ks2 = np.uint32(ks0 ^ ks1 ^ np.uint32(0x1BD11BDA))
    x0 = (x0 + ks0).astype(np.uint32)
    x1 = (x1 + ks1).astype(np.uint32)
    sched = ((ks1, ks2, 1), (ks2, ks0, 2), (ks0, ks1, 3),
             (ks1, ks2, 4), (ks2, ks0, 5))
    for i, (a, b, c) in enumerate(sched):
        for r in (rot_a if i % 2 == 0 else rot_b):
            x0 = (x0 + x1).astype(np.uint32)
            x1 = _rotl(x1, r)
            x1 = (x1 ^ x0).astype(np.uint32)
        x0 = (x0 + a).astype(np.uint32)
        x1 = (x1 + b + np.uint32(c)).astype(np.uint32)
    return x0, x1


def _uniform_draws(seed, size):
    k0 = np.uint32(seed >> 32)
    k1 = np.uint32(seed & 0xFFFFFFFF)
    c64 = np.arange(size, dtype=np.uint64)
    hi = (c64 >> np.uint64(32)).astype(np.uint32)
    lo = (c64 & np.uint64(0xFFFFFFFF)).astype(np.uint32)
    b0, b1 = _threefry2x32(k0, k1, hi, lo)
    bits = (b0 ^ b1).astype(np.uint32)
    fl = ((bits >> np.uint32(9)) | np.uint32(0x3F800000)).view(np.float32)
    return np.maximum(np.float32(0.0), fl - np.float32(1.0))


def _worker_layout(u_flat):
    u = u_flat.reshape(NS, NC, R2, P)
    return np.ascontiguousarray(u.transpose(0, 1, 3, 2)).reshape(-1)


_U_CONST = _worker_layout(_uniform_draws(42, B * P))


def kernel(x):
    return _masking_kernel(x, _U_CONST)


# speedup vs baseline: 1.1604x; 1.1604x over previous
import functools

import numpy as np

import jax
import jax.numpy as jnp
from jax import lax
from jax.experimental import pallas as pl
from jax.experimental.pallas import tpu as pltpu
from jax.experimental.pallas import tpu_sc as plsc

B = 4096
D = 56
P = 18
NBINS = P + 1
HIGH_PROB = 0.3
LOW_PROB = 0.05
ACT_THR = 0.01

NC = 2
NS = 16
R1 = B // NS
R2 = B // (NC * NS)
UW = R2 * P
RANK_LO = 3072
RANK_HI = 3073


def _sc_body(x_hbm, u_hbm, out_hbm, xbuf, ubuf, mbuf, abuf, cntbuf, gathbuf,
             shared, usem):
    c = lax.axis_index("c")
    s = lax.axis_index("s")
    row0 = s * R1
    prow = c * R2
    grow = row0 + prow

    ucopy = pltpu.make_async_copy(u_hbm.at[pl.ds((s * NC + c) * UW, UW)],
                                  ubuf, usem)
    ucopy.start()
    pltpu.sync_copy(x_hbm.at[pl.ds(row0, R1)], xbuf)

    iot = lax.iota(jnp.int32, 16)
    act_thr = jnp.float32(ACT_THR)
    colv = [jnp.full((16,), col, jnp.int32) for col in range(D)]

    acc = [jnp.zeros((16,), jnp.int32) for _ in range(NBINS)]
    for g in range(R1 // 16):
        rows = iot + g * 16
        m = jnp.zeros((16,), jnp.int32)
        actbits = jnp.zeros((16,), jnp.int32)
        for p in range(P):
            xv = plsc.load_gather(xbuf, [rows, colv[2 + 3 * p]])
            a = xv > act_thr
            m = m + a.astype(jnp.int32)
            actbits = actbits | jnp.where(a, jnp.int32(1 << p), 0)
        mbuf[pl.ds(g * 16, 16)] = m
        abuf[pl.ds(g * 16, 16)] = actbits
        for k in range(NBINS):
            acc[k] = acc[k] + plsc.all_reduce_population_count(m <= k)

    cnt_lo = jnp.zeros((16,), jnp.int32)
    cnt_hi = jnp.zeros((16,), jnp.int32)
    for k in range(16):
        cnt_lo = jnp.where(iot == k, acc[k], cnt_lo)
    for k in range(16, NBINS):
        cnt_hi = jnp.where(iot == (k - 16), acc[k], cnt_hi)
    cntbuf[pl.ds(0, 16)] = cnt_lo
    cntbuf[pl.ds(16, 16)] = cnt_hi

    pltpu.sync_copy(cntbuf, shared.at[pl.ds(s * 32, 32)])
    plsc.subcore_barrier()
    pltpu.sync_copy(shared, gathbuf)
    tot_lo = jnp.zeros((16,), jnp.int32)
    tot_hi = jnp.zeros((16,), jnp.int32)
    for t in range(NS):
        tot_lo = tot_lo + gathbuf[pl.ds(t * 32, 16)]
        tot_hi = tot_hi + gathbuf[pl.ds(t * 32 + 16, 16)]

    hi_valid = iot < (NBINS - 16)

    def order_stat(rank):
        lo = jnp.sum((tot_lo < rank).astype(jnp.int32))
        hi = jnp.sum(((tot_hi < rank) & hi_valid).astype(jnp.int32))
        return lo + hi

    s_lo = order_stat(RANK_LO)
    s_hi = order_stat(RANK_HI)
    thr = s_lo.astype(jnp.float32) + jnp.float32(0.25) * (
        s_hi - s_lo).astype(jnp.float32)

    high = jnp.float32(HIGH_PROB)
    low = jnp.float32(LOW_PROB)
    zero16 = jnp.zeros((16,), jnp.float32)
    ucopy.wait()
    for g in range(R2 // 16):
        xrows = iot + (g * 16) + prow
        m = mbuf[pl.ds(prow + g * 16, 16)]
        actbits = abuf[pl.ds(prow + g * 16, 16)]
        probv = jnp.where(m.astype(jnp.float32) > thr, high, low)
        for p in range(P):
            col = 2 + 3 * p
            uv = ubuf[pl.ds(p * R2 + g * 16, 16)]
            act = (actbits & jnp.int32(1 << p)) != 0
            mk = act & (uv < probv)
            for j in (0, 1, 2):
                plsc.store_scatter(xbuf, [xrows, colv[col + j]], zero16,
                                   mask=mk)

    pltpu.sync_copy(xbuf.at[pl.ds(prow, R2)], out_hbm.at[pl.ds(grow, R2)])


@functools.partial(
    pl.kernel,
    out_type=jax.ShapeDtypeStruct((B, D), jnp.float32),
    mesh=plsc.VectorSubcoreMesh(core_axis_name="c", subcore_axis_name="s",
                                num_cores=NC, num_subcores=NS),
    scratch_types=[
        pltpu.VMEM((R1, D), jnp.float32),
        pltpu.VMEM((UW,), jnp.float32),
        pltpu.VMEM((R1,), jnp.int32),
        pltpu.VMEM((R1,), jnp.int32),
        pltpu.VMEM((32,), jnp.int32),
        pltpu.VMEM((NS * 32,), jnp.int32),
        pltpu.VMEM_SHARED((NS * 32,), jnp.int32),
        pltpu.SemaphoreType.DMA,
    ],
    compiler_params=pltpu.CompilerParams(needs_layout_passes=False),
)
def _masking_kernel(x_hbm, u_hbm, out_hbm, xbuf, ubuf, mbuf, abuf, cntbuf,
                    gathbuf, shared, usem):
    _sc_body(x_hbm, u_hbm, out_hbm, xbuf, ubuf, mbuf, abuf, cntbuf, gathbuf,
             shared, usem)


def _rotl(x, r):
    return ((x << np.uint32(r)) | (x >> np.uint32(32 - r))).astype(np.uint32)


def _threefry2x32(k0, k1, x0, x1):
    rot_a = (13, 15, 26, 6)
    rot_b = (17, 29, 16, 24)
    ks0 = np.uint32(k0)
    ks1 = np.uint32(k1)
    ks2 = np.uint32(ks0 ^ ks1 ^ np.uint32(0x1BD11BDA))
    x0 = (x0 + ks0).astype(np.uint32)
    x1 = (x1 + ks1).astype(np.uint32)
    sched = ((ks1, ks2, 1), (ks2, ks0, 2), (ks0, ks1, 3),
             (ks1, ks2, 4), (ks2, ks0, 5))
    for i, (a, b, c) in enumerate(sched):
        for r in (rot_a if i % 2 == 0 else rot_b):
            x0 = (x0 + x1).astype(np.uint32)
            x1 = _rotl(x1, r)
            x1 = (x1 ^ x0).astype(np.uint32)
        x0 = (x0 + a).astype(np.uint32)
        x1 = (x1 + b + np.uint32(c)).astype(np.uint32)
    return x0, x1


def _uniform_draws(seed, size):
    k0 = np.uint32(seed >> 32)
    k1 = np.uint32(seed & 0xFFFFFFFF)
    c64 = np.arange(size, dtype=np.uint64)
    hi = (c64 >> np.uint64(32)).astype(np.uint32)
    lo = (c64 & np.uint64(0xFFFFFFFF)).astype(np.uint32)
    b0, b1 = _threefry2x32(k0, k1, hi, lo)
    bits = (b0 ^ b1).astype(np.uint32)
    fl = ((bits >> np.uint32(9)) | np.uint32(0x3F800000)).view(np.float32)
    return np.maximum(np.float32(0.0), fl - np.float32(1.0))


def _worker_layout(u_flat):
    u = u_flat.reshape(NS, NC, R2, P)
    return np.ascontiguousarray(u.transpose(0, 1, 3, 2)).reshape(-1)


_U_CONST = _worker_layout(_uniform_draws(42, B * P))


def kernel(x):
    return _masking_kernel(x, _U_CONST)
